# Initial kernel scaffold; baseline (speedup 1.0000x reference)
import jax
import jax.numpy as jnp
from jax.experimental import pallas as pl


def kernel(x, edge_index, batch, W0, b0, g0, be0, W1, b1, g1, be1, W2, b2, g2, be2, W3, b3, g3, be3, fW1, fb1, fW2, fb2, fW3, fb3):
    raise NotImplementedError("write your pallas kernel here")


# speedup vs baseline: 13.7087x; 13.7087x over previous
import functools

import jax
import jax.numpy as jnp
from jax import lax
from jax.experimental import pallas as pl
from jax.experimental.pallas import tpu as pltpu
from jax.experimental.pallas import tpu_sc as plsc

N = 10000
NPAD = 10240
E = 320000
EPAD = 327680
EROWS = EPAD // 128
F = 128
G = 128
EPS = 1e-5
NW = 32
EPW_ROWS = EROWS // NW
ACC_ROWS_W = NPAD // 16
BLK = 256
NB = NPAD // BLK
SEG_PER_W = G // NW
CH = 128

_mesh = plsc.VectorSubcoreMesh(core_axis_name="c", subcore_axis_name="s")


@functools.partial(
    pl.kernel,
    out_type=jax.ShapeDtypeStruct((2, NPAD), jnp.float32),
    mesh=_mesh,
    scratch_types=[
        pltpu.VMEM((EPW_ROWS, 128), jnp.int32),
        pltpu.VMEM((128,), jnp.float32),
        pltpu.VMEM((ACC_ROWS_W,), jnp.float32),
        pltpu.VMEM_SHARED((NPAD,), jnp.float32),
    ],
)
def _deg_kernel(dst_hbm, out_hbm, idx_v, ones_v, zbuf_v, acc):
    cid = lax.axis_index("c")
    sid = lax.axis_index("s")
    wid = cid * 16 + sid

    def fill_ones(i, _):
        ones_v[pl.ds(i * 16, 16)] = jnp.ones((16,), jnp.float32)
        return 0

    lax.fori_loop(0, 8, fill_ones, 0)

    def fill_zeros(i, _):
        zbuf_v[pl.ds(i * 16, 16)] = jnp.zeros((16,), jnp.float32)
        return 0

    lax.fori_loop(0, ACC_ROWS_W // 16, fill_zeros, 0)
    pltpu.sync_copy(zbuf_v, acc.at[pl.ds(sid * ACC_ROWS_W, ACC_ROWS_W)])
    plsc.subcore_barrier()

    pltpu.sync_copy(dst_hbm.at[pl.ds(wid * EPW_ROWS, EPW_ROWS)], idx_v)

    def body(j, _):
        pltpu.sync_copy(ones_v, acc.at[idx_v.at[j]], add=True)
        return 0

    lax.fori_loop(0, EPW_ROWS, body, 0)
    plsc.subcore_barrier()
    pltpu.sync_copy(acc.at[pl.ds(sid * ACC_ROWS_W, ACC_ROWS_W)],
                    out_hbm.at[cid, pl.ds(sid * ACC_ROWS_W, ACC_ROWS_W)])


@functools.partial(
    pl.kernel,
    out_type=jax.ShapeDtypeStruct((2, NPAD, F), jnp.float32),
    mesh=_mesh,
    scratch_types=[
        pltpu.VMEM((EPW_ROWS, 128), jnp.int32),
        pltpu.VMEM((EPW_ROWS, 128), jnp.int32),
        pltpu.VMEM((128, F), jnp.float32),
        pltpu.VMEM_SHARED((NPAD, F), jnp.float32),
        pltpu.SemaphoreType.DMA,
    ],
)
def _agg_kernel(hs_hbm, src_hbm, dst_hbm, out_hbm, src_v, dst_v, rows_v, acc, sem):
    cid = lax.axis_index("c")
    sid = lax.axis_index("s")
    wid = cid * 16 + sid

    def fill_zeros(i, _):
        r = i // 8
        cpos = (i % 8) * 16
        rows_v[r, pl.ds(cpos, 16)] = jnp.zeros((16,), jnp.float32)
        return 0

    lax.fori_loop(0, 128 * 8, fill_zeros, 0)

    def zcopy(i, _):
        pltpu.sync_copy(rows_v, acc.at[pl.ds(sid * ACC_ROWS_W + i * 128, 128)])
        return 0

    lax.fori_loop(0, ACC_ROWS_W // 128, zcopy, 0)
    plsc.subcore_barrier()

    pltpu.sync_copy(src_hbm.at[pl.ds(wid * EPW_ROWS, EPW_ROWS)], src_v)
    pltpu.sync_copy(dst_hbm.at[pl.ds(wid * EPW_ROWS, EPW_ROWS)], dst_v)

    def body(j, _):
        pltpu.async_copy(hs_hbm.at[src_v.at[j]], rows_v, sem).wait()
        pltpu.sync_copy(rows_v, acc.at[dst_v.at[j]], add=True)
        return 0

    lax.fori_loop(0, EPW_ROWS, body, 0)
    plsc.subcore_barrier()
    pltpu.sync_copy(acc.at[pl.ds(sid * ACC_ROWS_W, ACC_ROWS_W)],
                    out_hbm.at[cid, pl.ds(sid * ACC_ROWS_W, ACC_ROWS_W)])


@functools.partial(
    pl.kernel,
    out_type=[jax.ShapeDtypeStruct((G, F), jnp.float32),
              jax.ShapeDtypeStruct((G, F), jnp.float32)],
    mesh=_mesh,
    scratch_types=[
        pltpu.VMEM((NW * 16,), jnp.int32),
        pltpu.VMEM((NW * 16,), jnp.int32),
        pltpu.VMEM((CH, F), jnp.float32),
        pltpu.VMEM((F,), jnp.float32),
        pltpu.VMEM((F,), jnp.float32),
    ],
)
def _pool_kernel(h_hbm, starts_hbm, counts_hbm, osum_hbm, omax_hbm,
                 st_s, ct_s, rows_v, obuf_s, obuf_m):
    cid = lax.axis_index("c")
    sid = lax.axis_index("s")
    wid = cid * 16 + sid
    pltpu.sync_copy(starts_hbm, st_s)
    pltpu.sync_copy(counts_hbm, ct_s)
    vst = st_s[pl.ds(wid * 16, 16)]
    vct = ct_s[pl.ds(wid * 16, 16)]
    for k in range(SEG_PER_W):
        g = wid * SEG_PER_W + k
        st = vst[k]
        cnt = vct[k]
        st8 = (st // 8) * 8
        off = st - st8
        nch = (off + cnt + CH - 1) // CH
        init = tuple(jnp.zeros((16,), jnp.float32) for _ in range(16))

        def chunk_body(c, carry):
            pltpu.sync_copy(h_hbm.at[pl.ds(st8 + c * CH, CH)], rows_v)
            lo = jnp.maximum(off - c * CH, 0)
            hi = jnp.minimum(off + cnt - c * CH, CH)

            def row_body(r, rc):
                vals = [rows_v[r, pl.ds(q * 16, 16)] for q in range(8)]
                new = [rc[q] + vals[q] for q in range(8)]
                new += [jnp.maximum(rc[8 + q], vals[q]) for q in range(8)]
                return tuple(new)

            return lax.fori_loop(lo, hi, row_body, carry)

        res = lax.fori_loop(0, nch, chunk_body, init)
        for q in range(8):
            obuf_s[pl.ds(q * 16, 16)] = res[q]
            obuf_m[pl.ds(q * 16, 16)] = res[8 + q]
        pltpu.sync_copy(obuf_s, osum_hbm.at[g])
        pltpu.sync_copy(obuf_m, omax_hbm.at[g])


def _pre_body(xb, w0, d0b, d1b, bcolb, dis_out, hs_out, cf_out, si_out, ci_out, cacc):
    i = pl.program_id(0)

    @pl.when(i == 0)
    def _():
        cacc[...] = jnp.zeros_like(cacc)

    cnt = d0b[...] + d1b[...]
    dis = lax.rsqrt(cnt + 1.0)
    dis_out[...] = dis
    hs_out[...] = jnp.dot(xb[...], w0[...], preferred_element_type=jnp.float32) * dis
    lanes = lax.broadcasted_iota(jnp.int32, (BLK, G), 1)
    onehot = (bcolb[...] == lanes).astype(jnp.float32)
    cacc[...] += jnp.sum(onehot, axis=0, keepdims=True)

    @pl.when(i == NB - 1)
    def _():
        crow = cacc[...]
        cf_out[...] = crow
        jj = lax.broadcasted_iota(jnp.int32, (G, G), 0)
        gg = lax.broadcasted_iota(jnp.int32, (G, G), 1)
        tri = (jj < gg).astype(jnp.float32)
        starts = jnp.dot(crow, tri, preferred_element_type=jnp.float32)
        si_out[...] = starts.astype(jnp.int32)
        ci_out[...] = crow.astype(jnp.int32)


_pre_call = pl.pallas_call(
    _pre_body,
    grid=(NB,),
    in_specs=[
        pl.BlockSpec((BLK, F), lambda i: (i, 0)),
        pl.BlockSpec((F, F), lambda i: (0, 0)),
        pl.BlockSpec((BLK, 1), lambda i: (i, 0)),
        pl.BlockSpec((BLK, 1), lambda i: (i, 0)),
        pl.BlockSpec((BLK, 1), lambda i: (i, 0)),
    ],
    out_specs=[
        pl.BlockSpec((BLK, 1), lambda i: (i, 0)),
        pl.BlockSpec((BLK, F), lambda i: (i, 0)),
        pl.BlockSpec((1, G), lambda i: (0, 0)),
        pl.BlockSpec((1, G), lambda i: (0, 0)),
        pl.BlockSpec((1, G), lambda i: (0, 0)),
    ],
    out_shape=[
        jax.ShapeDtypeStruct((NPAD, 1), jnp.float32),
        jax.ShapeDtypeStruct((NPAD, F), jnp.float32),
        jax.ShapeDtypeStruct((1, G), jnp.float32),
        jax.ShapeDtypeStruct((1, G), jnp.int32),
        jax.ShapeDtypeStruct((1, G), jnp.int32),
    ],
    scratch_shapes=[pltpu.VMEM((1, G), jnp.float32)],
)


def _make_layer_call(has_next):
    def body(p0b, p1b, hsb, disb, bvec, gvec, bevec, wn, out, ssum, ssq):
        ph = pl.program_id(0)
        i = pl.program_id(1)
        t = disb[...] * (p0b[...] + p1b[...] + hsb[...]) + bvec[...]
        rows = lax.broadcasted_iota(jnp.int32, (BLK, 1), 0) + i * BLK
        msk = (rows < N).astype(jnp.float32)

        @pl.when(ph == 0)
        def _():
            @pl.when(i == 0)
            def _():
                ssum[...] = jnp.zeros_like(ssum)
                ssq[...] = jnp.zeros_like(ssq)

            tm = t * msk
            ssum[...] += jnp.sum(tm, axis=0, keepdims=True)
            ssq[...] += jnp.sum(tm * t, axis=0, keepdims=True)

        @pl.when(ph == 1)
        def _():
            m = ssum[...] * (1.0 / N)
            v = ssq[...] * (1.0 / N) - m * m
            hb = jnp.maximum((t - m) * lax.rsqrt(v + EPS) * gvec[...] + bevec[...], 0.0) * msk
            if has_next:
                out[...] = jnp.dot(hb, wn[...], preferred_element_type=jnp.float32) * disb[...]
            else:
                out[...] = hb

    return pl.pallas_call(
        body,
        grid=(2, NB),
        in_specs=[
            pl.BlockSpec((BLK, F), lambda p, i: (i, 0)),
            pl.BlockSpec((BLK, F), lambda p, i: (i, 0)),
            pl.BlockSpec((BLK, F), lambda p, i: (i, 0)),
            pl.BlockSpec((BLK, 1), lambda p, i: (i, 0)),
            pl.BlockSpec((1, F), lambda p, i: (0, 0)),
            pl.BlockSpec((1, F), lambda p, i: (0, 0)),
            pl.BlockSpec((1, F), lambda p, i: (0, 0)),
            pl.BlockSpec((F, F), lambda p, i: (0, 0)),
        ],
        out_specs=pl.BlockSpec((BLK, F), lambda p, i: (i, 0)),
        out_shape=jax.ShapeDtypeStruct((NPAD, F), jnp.float32),
        scratch_shapes=[pltpu.VMEM((1, F), jnp.float32),
                        pltpu.VMEM((1, F), jnp.float32)],
    )


_layer_call = _make_layer_call(True)
_layer_last_call = _make_layer_call(False)


def _head_body(osum, omax, cfrow, fw1, fb1, fw2, fb2, fw3, fb3, out):
    cnt = cfrow[...]
    rcp = 1.0 / jnp.maximum(cnt, 1.0)
    pos = (cnt > 0.0).astype(jnp.float32)
    ii = lax.broadcasted_iota(jnp.int32, (G, G), 0)
    jj = lax.broadcasted_iota(jnp.int32, (G, G), 1)
    eye = (ii == jj).astype(jnp.float32)
    s = osum[...]
    mean = jnp.dot(eye * rcp, s, preferred_element_type=jnp.float32)
    mx = jnp.dot(eye * pos, omax[...], preferred_element_type=jnp.float32)
    w1 = fw1[...]
    z1 = (jnp.dot(mean, w1[0:G], preferred_element_type=jnp.float32)
          + jnp.dot(mx, w1[G:2 * G], preferred_element_type=jnp.float32)
          + jnp.dot(s, w1[2 * G:3 * G], preferred_element_type=jnp.float32)
          + fb1[...])
    z1 = jnp.maximum(z1, 0.0)
    z2 = jnp.maximum(jnp.dot(z1, fw2[...], preferred_element_type=jnp.float32) + fb2[...], 0.0)
    out[...] = jnp.dot(z2, fw3[...], preferred_element_type=jnp.float32) + fb3[...]


_head_call = pl.pallas_call(
    _head_body,
    out_shape=jax.ShapeDtypeStruct((G, 1), jnp.float32),
)


def kernel(x, edge_index, batch, W0, b0, g0, be0, W1, b1, g1, be1, W2, b2, g2, be2,
           W3, b3, g3, be3, fW1, fb1, fW2, fb2, fW3, fb3):
    xp = jnp.zeros((NPAD, F), jnp.float32).at[:N].set(x)
    pad = N + (jnp.arange(EPAD - E, dtype=jnp.int32) % 128)
    src2d = jnp.concatenate([edge_index[0], pad]).reshape(EROWS, 128)
    dst2d = jnp.concatenate([edge_index[1], pad]).reshape(EROWS, 128)
    bcol = jnp.concatenate([batch, jnp.full((NPAD - N,), G, jnp.int32)]).reshape(NPAD, 1)

    degp = _deg_kernel(dst2d)
    d0 = degp[0].reshape(NPAD, 1)
    d1 = degp[1].reshape(NPAD, 1)

    dis, hs, cf, si, ci = _pre_call(xp, W0, d0, d1, bcol)

    layers = [(b0, g0, be0, W1), (b1, g1, be1, W2), (b2, g2, be2, W3), (b3, g3, be3, W3)]
    for i, (bi, gi, bei, wn) in enumerate(layers):
        part = _agg_kernel(hs, src2d, dst2d)
        call = _layer_call if i < 3 else _layer_last_call
        hs = call(part[0], part[1], hs, dis,
                  bi.reshape(1, F), gi.reshape(1, F), bei.reshape(1, F), wn)

    si512 = jnp.pad(si.reshape(NW, SEG_PER_W), ((0, 0), (0, 16 - SEG_PER_W))).reshape(NW * 16)
    ci512 = jnp.pad(ci.reshape(NW, SEG_PER_W), ((0, 0), (0, 16 - SEG_PER_W))).reshape(NW * 16)
    osum, omax = _pool_kernel(hs, si512, ci512)
    return _head_call(osum, omax, cf, fW1, fb1.reshape(1, F),
                      fW2, fb2.reshape(1, F // 2), fW3, fb3.reshape(1, 1))


# speedup vs baseline: 18.3089x; 1.3356x over previous
import functools

import jax
import jax.numpy as jnp
from jax import lax
from jax.experimental import pallas as pl
from jax.experimental.pallas import tpu as pltpu
from jax.experimental.pallas import tpu_sc as plsc

N = 10000
NPAD = 10240
E = 320000
EPAD = 327680
CHK = 128
EROWS = EPAD // CHK
F = 128
G = 128
EPS = 1e-5
NW = 32
EPW_ROWS = EROWS // NW
WIN = 40
ACC_ROWS_W = NPAD // 16
BLK = 256
NB = NPAD // BLK
SEG_PER_W = G // NW
CH = 128

_mesh = plsc.VectorSubcoreMesh(core_axis_name="c", subcore_axis_name="s")


@functools.partial(
    pl.kernel,
    out_type=jax.ShapeDtypeStruct((2, NPAD), jnp.float32),
    mesh=_mesh,
    scratch_types=[
        pltpu.VMEM((EPW_ROWS, CHK), jnp.int32),
        pltpu.VMEM((CHK,), jnp.float32),
        pltpu.VMEM((ACC_ROWS_W,), jnp.float32),
        pltpu.VMEM_SHARED((NPAD,), jnp.float32),
    ],
)
def _deg_kernel(dst_hbm, out_hbm, idx_v, ones_v, zbuf_v, acc):
    cid = lax.axis_index("c")
    sid = lax.axis_index("s")
    wid = cid * 16 + sid

    def fill_ones(i, _):
        ones_v[pl.ds(i * 16, 16)] = jnp.ones((16,), jnp.float32)
        return 0

    lax.fori_loop(0, CHK // 16, fill_ones, 0)

    def fill_zeros(i, _):
        zbuf_v[pl.ds(i * 16, 16)] = jnp.zeros((16,), jnp.float32)
        return 0

    lax.fori_loop(0, ACC_ROWS_W // 16, fill_zeros, 0)
    pltpu.sync_copy(zbuf_v, acc.at[pl.ds(sid * ACC_ROWS_W, ACC_ROWS_W)])
    plsc.subcore_barrier()

    pltpu.sync_copy(dst_hbm.at[pl.ds(wid * EPW_ROWS, EPW_ROWS)], idx_v)

    def body(j, _):
        pltpu.sync_copy(ones_v, acc.at[idx_v.at[j]], add=True)
        return 0

    lax.fori_loop(0, EPW_ROWS, body, 0)
    plsc.subcore_barrier()
    pltpu.sync_copy(acc.at[pl.ds(sid * ACC_ROWS_W, ACC_ROWS_W)],
                    out_hbm.at[cid, pl.ds(sid * ACC_ROWS_W, ACC_ROWS_W)])


@functools.partial(
    pl.kernel,
    out_type=jax.ShapeDtypeStruct((2, NPAD, F), jnp.float32),
    mesh=_mesh,
    scratch_types=[
        pltpu.VMEM((WIN, CHK), jnp.int32),
        pltpu.VMEM((WIN, CHK), jnp.int32),
        pltpu.VMEM((CHK, F), jnp.float32),
        pltpu.VMEM((CHK, F), jnp.float32),
        pltpu.VMEM_SHARED((NPAD, F), jnp.float32),
        pltpu.SemaphoreType.DMA,
        pltpu.SemaphoreType.DMA,
        pltpu.SemaphoreType.DMA,
        pltpu.SemaphoreType.DMA,
    ],
)
def _agg_kernel(hs_hbm, src_hbm, dst_hbm, out_hbm, src_v, dst_v, row_a, row_b,
                acc, gsa, gsb, ssa, ssb):
    cid = lax.axis_index("c")
    sid = lax.axis_index("s")
    wid = cid * 16 + sid

    def fill_zeros(i, _):
        r = i // 8
        cpos = (i % 8) * 16
        row_a[r, pl.ds(cpos, 16)] = jnp.zeros((16,), jnp.float32)
        return 0

    lax.fori_loop(0, CHK * 8, fill_zeros, 0)

    def zcopy(i, _):
        pltpu.sync_copy(row_a, acc.at[pl.ds(sid * ACC_ROWS_W + i * CHK, CHK)])
        return 0

    lax.fori_loop(0, ACC_ROWS_W // CHK, zcopy, 0)
    plsc.subcore_barrier()

    def wait_g(buf, sem):
        pltpu.make_async_copy(hs_hbm.at[src_v.at[0]], buf, sem).wait()

    def wait_s(buf, sem):
        pltpu.make_async_copy(buf, acc.at[dst_v.at[0]], sem).wait()

    npair = WIN // 2
    for ph in range(EPW_ROWS // WIN):
        pltpu.sync_copy(src_hbm.at[pl.ds(wid * EPW_ROWS + ph * WIN, WIN)], src_v)
        pltpu.sync_copy(dst_hbm.at[pl.ds(wid * EPW_ROWS + ph * WIN, WIN)], dst_v)
        pltpu.async_copy(hs_hbm.at[src_v.at[0]], row_a, gsa)

        def body(j2, _):
            j = 2 * j2
            pltpu.async_copy(hs_hbm.at[src_v.at[j + 1]], row_b, gsb)
            wait_g(row_a, gsa)
            pltpu.async_copy(row_a, acc.at[dst_v.at[j]], ssa, add=True)
            wait_s(row_a, ssa)

            @pl.when(j2 < npair - 1)
            def _():
                pltpu.async_copy(hs_hbm.at[src_v.at[j + 2]], row_a, gsa)

            wait_g(row_b, gsb)
            pltpu.async_copy(row_b, acc.at[dst_v.at[j + 1]], ssb, add=True)
            wait_s(row_b, ssb)
            return 0

        lax.fori_loop(0, npair, body, 0)
    plsc.subcore_barrier()
    pltpu.sync_copy(acc.at[pl.ds(sid * ACC_ROWS_W, ACC_ROWS_W)],
                    out_hbm.at[cid, pl.ds(sid * ACC_ROWS_W, ACC_ROWS_W)])


@functools.partial(
    pl.kernel,
    out_type=[jax.ShapeDtypeStruct((G, F), jnp.float32),
              jax.ShapeDtypeStruct((G, F), jnp.float32)],
    mesh=_mesh,
    scratch_types=[
        pltpu.VMEM((NW * 16,), jnp.int32),
        pltpu.VMEM((NW * 16,), jnp.int32),
        pltpu.VMEM((CH, F), jnp.float32),
        pltpu.VMEM((F,), jnp.float32),
        pltpu.VMEM((F,), jnp.float32),
    ],
)
def _pool_kernel(h_hbm, starts_hbm, counts_hbm, osum_hbm, omax_hbm,
                 st_s, ct_s, rows_v, obuf_s, obuf_m):
    cid = lax.axis_index("c")
    sid = lax.axis_index("s")
    wid = cid * 16 + sid
    pltpu.sync_copy(starts_hbm, st_s)
    pltpu.sync_copy(counts_hbm, ct_s)
    vst = st_s[pl.ds(wid * 16, 16)]
    vct = ct_s[pl.ds(wid * 16, 16)]
    for k in range(SEG_PER_W):
        g = wid * SEG_PER_W + k
        st = vst[k]
        cnt = vct[k]
        st8 = (st // 8) * 8
        off = st - st8
        nch = (off + cnt + CH - 1) // CH
        init = tuple(jnp.zeros((16,), jnp.float32) for _ in range(16))

        def chunk_body(c, carry):
            pltpu.sync_copy(h_hbm.at[pl.ds(st8 + c * CH, CH)], rows_v)
            lo = jnp.maximum(off - c * CH, 0)
            hi = jnp.minimum(off + cnt - c * CH, CH)

            def row_body(r, rc):
                vals = [rows_v[r, pl.ds(q * 16, 16)] for q in range(8)]
                new = [rc[q] + vals[q] for q in range(8)]
                new += [jnp.maximum(rc[8 + q], vals[q]) for q in range(8)]
                return tuple(new)

            return lax.fori_loop(lo, hi, row_body, carry)

        res = lax.fori_loop(0, nch, chunk_body, init)
        for q in range(8):
            obuf_s[pl.ds(q * 16, 16)] = res[q]
            obuf_m[pl.ds(q * 16, 16)] = res[8 + q]
        pltpu.sync_copy(obuf_s, osum_hbm.at[g])
        pltpu.sync_copy(obuf_m, omax_hbm.at[g])


def _pre_body(xb, w0, d0b, d1b, bcolb, dis_out, hs_out, cf_out, si_out, ci_out, cacc):
    i = pl.program_id(0)

    @pl.when(i == 0)
    def _():
        cacc[...] = jnp.zeros_like(cacc)

    cnt = d0b[...] + d1b[...]
    dis = lax.rsqrt(cnt + 1.0)
    dis_out[...] = dis
    hs_out[...] = jnp.dot(xb[...], w0[...], preferred_element_type=jnp.float32) * dis
    lanes = lax.broadcasted_iota(jnp.int32, (BLK, G), 1)
    onehot = (bcolb[...] == lanes).astype(jnp.float32)
    cacc[...] += jnp.sum(onehot, axis=0, keepdims=True)

    @pl.when(i == NB - 1)
    def _():
        crow = cacc[...]
        cf_out[...] = crow
        jj = lax.broadcasted_iota(jnp.int32, (G, G), 0)
        gg = lax.broadcasted_iota(jnp.int32, (G, G), 1)
        tri = (jj < gg).astype(jnp.float32)
        starts = jnp.dot(crow, tri, preferred_element_type=jnp.float32)
        si_out[...] = starts.astype(jnp.int32)
        ci_out[...] = crow.astype(jnp.int32)


_pre_call = pl.pallas_call(
    _pre_body,
    grid=(NB,),
    in_specs=[
        pl.BlockSpec((BLK, F), lambda i: (i, 0)),
        pl.BlockSpec((F, F), lambda i: (0, 0)),
        pl.BlockSpec((BLK, 1), lambda i: (i, 0)),
        pl.BlockSpec((BLK, 1), lambda i: (i, 0)),
        pl.BlockSpec((BLK, 1), lambda i: (i, 0)),
    ],
    out_specs=[
        pl.BlockSpec((BLK, 1), lambda i: (i, 0)),
        pl.BlockSpec((BLK, F), lambda i: (i, 0)),
        pl.BlockSpec((1, G), lambda i: (0, 0)),
        pl.BlockSpec((1, G), lambda i: (0, 0)),
        pl.BlockSpec((1, G), lambda i: (0, 0)),
    ],
    out_shape=[
        jax.ShapeDtypeStruct((NPAD, 1), jnp.float32),
        jax.ShapeDtypeStruct((NPAD, F), jnp.float32),
        jax.ShapeDtypeStruct((1, G), jnp.float32),
        jax.ShapeDtypeStruct((1, G), jnp.int32),
        jax.ShapeDtypeStruct((1, G), jnp.int32),
    ],
    scratch_shapes=[pltpu.VMEM((1, G), jnp.float32)],
)


def _make_layer_call(has_next):
    def body(p0b, p1b, hsb, disb, bvec, gvec, bevec, wn, out, ssum, ssq):
        ph = pl.program_id(0)
        i = pl.program_id(1)
        t = disb[...] * (p0b[...] + p1b[...] + hsb[...]) + bvec[...]
        rows = lax.broadcasted_iota(jnp.int32, (BLK, 1), 0) + i * BLK
        msk = (rows < N).astype(jnp.float32)

        @pl.when(ph == 0)
        def _():
            @pl.when(i == 0)
            def _():
                ssum[...] = jnp.zeros_like(ssum)
                ssq[...] = jnp.zeros_like(ssq)

            tm = t * msk
            ssum[...] += jnp.sum(tm, axis=0, keepdims=True)
            ssq[...] += jnp.sum(tm * t, axis=0, keepdims=True)

        @pl.when(ph == 1)
        def _():
            m = ssum[...] * (1.0 / N)
            v = ssq[...] * (1.0 / N) - m * m
            hb = jnp.maximum((t - m) * lax.rsqrt(v + EPS) * gvec[...] + bevec[...], 0.0) * msk
            if has_next:
                out[...] = jnp.dot(hb, wn[...], preferred_element_type=jnp.float32) * disb[...]
            else:
                out[...] = hb

    return pl.pallas_call(
        body,
        grid=(2, NB),
        in_specs=[
            pl.BlockSpec((BLK, F), lambda p, i: (i, 0)),
            pl.BlockSpec((BLK, F), lambda p, i: (i, 0)),
            pl.BlockSpec((BLK, F), lambda p, i: (i, 0)),
            pl.BlockSpec((BLK, 1), lambda p, i: (i, 0)),
            pl.BlockSpec((1, F), lambda p, i: (0, 0)),
            pl.BlockSpec((1, F), lambda p, i: (0, 0)),
            pl.BlockSpec((1, F), lambda p, i: (0, 0)),
            pl.BlockSpec((F, F), lambda p, i: (0, 0)),
        ],
        out_specs=pl.BlockSpec((BLK, F), lambda p, i: (i, 0)),
        out_shape=jax.ShapeDtypeStruct((NPAD, F), jnp.float32),
        scratch_shapes=[pltpu.VMEM((1, F), jnp.float32),
                        pltpu.VMEM((1, F), jnp.float32)],
    )


_layer_call = _make_layer_call(True)
_layer_last_call = _make_layer_call(False)


def _head_body(osum, omax, cfrow, fw1, fb1, fw2, fb2, fw3, fb3, out):
    cnt = cfrow[...]
    rcp = 1.0 / jnp.maximum(cnt, 1.0)
    pos = (cnt > 0.0).astype(jnp.float32)
    ii = lax.broadcasted_iota(jnp.int32, (G, G), 0)
    jj = lax.broadcasted_iota(jnp.int32, (G, G), 1)
    eye = (ii == jj).astype(jnp.float32)
    s = osum[...]
    mean = jnp.dot(eye * rcp, s, preferred_element_type=jnp.float32)
    mx = jnp.dot(eye * pos, omax[...], preferred_element_type=jnp.float32)
    w1 = fw1[...]
    z1 = (jnp.dot(mean, w1[0:G], preferred_element_type=jnp.float32)
          + jnp.dot(mx, w1[G:2 * G], preferred_element_type=jnp.float32)
          + jnp.dot(s, w1[2 * G:3 * G], preferred_element_type=jnp.float32)
          + fb1[...])
    z1 = jnp.maximum(z1, 0.0)
    z2 = jnp.maximum(jnp.dot(z1, fw2[...], preferred_element_type=jnp.float32) + fb2[...], 0.0)
    out[...] = jnp.dot(z2, fw3[...], preferred_element_type=jnp.float32) + fb3[...]


_head_call = pl.pallas_call(
    _head_body,
    out_shape=jax.ShapeDtypeStruct((G, 1), jnp.float32),
)


def kernel(x, edge_index, batch, W0, b0, g0, be0, W1, b1, g1, be1, W2, b2, g2, be2,
           W3, b3, g3, be3, fW1, fb1, fW2, fb2, fW3, fb3):
    xp = jnp.zeros((NPAD, F), jnp.float32).at[:N].set(x)
    pad = N + (jnp.arange(EPAD - E, dtype=jnp.int32) % 128)
    src2d = jnp.concatenate([edge_index[0], pad]).reshape(EROWS, CHK)
    dst2d = jnp.concatenate([edge_index[1], pad]).reshape(EROWS, CHK)
    bcol = jnp.concatenate([batch, jnp.full((NPAD - N,), G, jnp.int32)]).reshape(NPAD, 1)

    degp = _deg_kernel(dst2d)
    d0 = degp[0].reshape(NPAD, 1)
    d1 = degp[1].reshape(NPAD, 1)

    dis, hs, cf, si, ci = _pre_call(xp, W0, d0, d1, bcol)

    layers = [(b0, g0, be0, W1), (b1, g1, be1, W2), (b2, g2, be2, W3), (b3, g3, be3, W3)]
    for i, (bi, gi, bei, wn) in enumerate(layers):
        part = _agg_kernel(hs, src2d, dst2d)
        call = _layer_call if i < 3 else _layer_last_call
        hs = call(part[0], part[1], hs, dis,
                  bi.reshape(1, F), gi.reshape(1, F), bei.reshape(1, F), wn)

    si512 = jnp.pad(si.reshape(NW, SEG_PER_W), ((0, 0), (0, 16 - SEG_PER_W))).reshape(NW * 16)
    ci512 = jnp.pad(ci.reshape(NW, SEG_PER_W), ((0, 0), (0, 16 - SEG_PER_W))).reshape(NW * 16)
    osum, omax = _pool_kernel(hs, si512, ci512)
    return _head_call(osum, omax, cf, fW1, fb1.reshape(1, F),
                      fW2, fb2.reshape(1, F // 2), fW3, fb3.reshape(1, 1))


# speedup vs baseline: 22.4840x; 1.2280x over previous
import functools

import jax
import jax.numpy as jnp
from jax import lax
from jax.experimental import pallas as pl
from jax.experimental.pallas import tpu as pltpu
from jax.experimental.pallas import tpu_sc as plsc

N = 10000
NPAD = 10240
E = 320000
EPAD = 327680
CHK = 128
EROWS = EPAD // CHK
F = 128
G = 128
EPS = 1e-5
NW = 32
EPW_ROWS = EROWS // NW
WIN = 40
ACC_ROWS_W = NPAD // 16
BLK = 1024
NB = NPAD // BLK
SEG_PER_W = G // NW
CH = 128

_mesh = plsc.VectorSubcoreMesh(core_axis_name="c", subcore_axis_name="s")


@functools.partial(
    pl.kernel,
    out_type=jax.ShapeDtypeStruct((2, NPAD), jnp.float32),
    mesh=_mesh,
    scratch_types=[
        pltpu.VMEM((EPW_ROWS, CHK), jnp.int32),
        pltpu.VMEM((CHK,), jnp.float32),
        pltpu.VMEM((ACC_ROWS_W,), jnp.float32),
        pltpu.VMEM_SHARED((NPAD,), jnp.float32),
    ],
)
def _deg_kernel(dst_hbm, out_hbm, idx_v, ones_v, zbuf_v, acc):
    cid = lax.axis_index("c")
    sid = lax.axis_index("s")
    wid = cid * 16 + sid

    def fill_ones(i, _):
        ones_v[pl.ds(i * 16, 16)] = jnp.ones((16,), jnp.float32)
        return 0

    lax.fori_loop(0, CHK // 16, fill_ones, 0)

    def fill_zeros(i, _):
        zbuf_v[pl.ds(i * 16, 16)] = jnp.zeros((16,), jnp.float32)
        return 0

    lax.fori_loop(0, ACC_ROWS_W // 16, fill_zeros, 0)
    pltpu.sync_copy(zbuf_v, acc.at[pl.ds(sid * ACC_ROWS_W, ACC_ROWS_W)])
    plsc.subcore_barrier()

    pltpu.sync_copy(dst_hbm.at[pl.ds(wid * EPW_ROWS, EPW_ROWS)], idx_v)

    def body(j, _):
        pltpu.sync_copy(ones_v, acc.at[idx_v.at[j]], add=True)
        return 0

    lax.fori_loop(0, EPW_ROWS, body, 0)
    plsc.subcore_barrier()
    pltpu.sync_copy(acc.at[pl.ds(sid * ACC_ROWS_W, ACC_ROWS_W)],
                    out_hbm.at[cid, pl.ds(sid * ACC_ROWS_W, ACC_ROWS_W)])


@functools.partial(
    pl.kernel,
    out_type=jax.ShapeDtypeStruct((2, NPAD, F), jnp.float32),
    mesh=_mesh,
    scratch_types=[
        pltpu.VMEM((WIN, CHK), jnp.int32),
        pltpu.VMEM((WIN, CHK), jnp.int32),
        pltpu.VMEM((CHK, F), jnp.float32),
        pltpu.VMEM((CHK, F), jnp.float32),
        pltpu.VMEM_SHARED((NPAD, F), jnp.float32),
        pltpu.SemaphoreType.DMA,
        pltpu.SemaphoreType.DMA,
        pltpu.SemaphoreType.DMA,
        pltpu.SemaphoreType.DMA,
    ],
)
def _agg_kernel(hs_hbm, src_hbm, dst_hbm, out_hbm, src_v, dst_v, row_a, row_b,
                acc, gsa, gsb, ssa, ssb):
    cid = lax.axis_index("c")
    sid = lax.axis_index("s")
    wid = cid * 16 + sid

    def fill_zeros(i, _):
        r = i // 8
        cpos = (i % 8) * 16
        row_a[r, pl.ds(cpos, 16)] = jnp.zeros((16,), jnp.float32)
        return 0

    lax.fori_loop(0, CHK * 8, fill_zeros, 0)

    def zcopy(i, _):
        pltpu.sync_copy(row_a, acc.at[pl.ds(sid * ACC_ROWS_W + i * CHK, CHK)])
        return 0

    lax.fori_loop(0, ACC_ROWS_W // CHK, zcopy, 0)
    plsc.subcore_barrier()

    def wait_g(buf, sem):
        pltpu.make_async_copy(hs_hbm.at[src_v.at[0]], buf, sem).wait()

    def wait_s(buf, sem):
        pltpu.make_async_copy(buf, acc.at[dst_v.at[0]], sem).wait()

    npair = WIN // 2
    for ph in range(EPW_ROWS // WIN):
        pltpu.sync_copy(src_hbm.at[pl.ds(wid * EPW_ROWS + ph * WIN, WIN)], src_v)
        pltpu.sync_copy(dst_hbm.at[pl.ds(wid * EPW_ROWS + ph * WIN, WIN)], dst_v)
        pltpu.async_copy(hs_hbm.at[src_v.at[0]], row_a, gsa)

        def body(j2, _):
            j = 2 * j2
            pltpu.async_copy(hs_hbm.at[src_v.at[j + 1]], row_b, gsb)
            wait_g(row_a, gsa)
            pltpu.async_copy(row_a, acc.at[dst_v.at[j]], ssa, add=True)
            wait_s(row_a, ssa)

            @pl.when(j2 < npair - 1)
            def _():
                pltpu.async_copy(hs_hbm.at[src_v.at[j + 2]], row_a, gsa)

            wait_g(row_b, gsb)
            pltpu.async_copy(row_b, acc.at[dst_v.at[j + 1]], ssb, add=True)
            wait_s(row_b, ssb)
            return 0

        lax.fori_loop(0, npair, body, 0)
    plsc.subcore_barrier()
    pltpu.sync_copy(acc.at[pl.ds(sid * ACC_ROWS_W, ACC_ROWS_W)],
                    out_hbm.at[cid, pl.ds(sid * ACC_ROWS_W, ACC_ROWS_W)])


@functools.partial(
    pl.kernel,
    out_type=[jax.ShapeDtypeStruct((G, F), jnp.float32),
              jax.ShapeDtypeStruct((G, F), jnp.float32)],
    mesh=_mesh,
    scratch_types=[
        pltpu.VMEM((NW * 16,), jnp.int32),
        pltpu.VMEM((NW * 16,), jnp.int32),
        pltpu.VMEM((CH, F), jnp.float32),
        pltpu.VMEM((F,), jnp.float32),
        pltpu.VMEM((F,), jnp.float32),
    ],
)
def _pool_kernel(h_hbm, starts_hbm, counts_hbm, osum_hbm, omax_hbm,
                 st_s, ct_s, rows_v, obuf_s, obuf_m):
    cid = lax.axis_index("c")
    sid = lax.axis_index("s")
    wid = cid * 16 + sid
    pltpu.sync_copy(starts_hbm, st_s)
    pltpu.sync_copy(counts_hbm, ct_s)
    vst = st_s[pl.ds(wid * 16, 16)]
    vct = ct_s[pl.ds(wid * 16, 16)]
    for k in range(SEG_PER_W):
        g = wid * SEG_PER_W + k
        st = vst[k]
        cnt = vct[k]
        st8 = (st // 8) * 8
        off = st - st8
        nch = (off + cnt + CH - 1) // CH
        init = tuple(jnp.zeros((16,), jnp.float32) for _ in range(16))

        def chunk_body(c, carry):
            pltpu.sync_copy(h_hbm.at[pl.ds(st8 + c * CH, CH)], rows_v)
            lo = jnp.maximum(off - c * CH, 0)
            hi = jnp.minimum(off + cnt - c * CH, CH)

            def row_body(r, rc):
                vals = [rows_v[r, pl.ds(q * 16, 16)] for q in range(8)]
                new = [rc[q] + vals[q] for q in range(8)]
                new += [jnp.maximum(rc[8 + q], vals[q]) for q in range(8)]
                return tuple(new)

            return lax.fori_loop(lo, hi, row_body, carry)

        res = lax.fori_loop(0, nch, chunk_body, init)
        for q in range(8):
            obuf_s[pl.ds(q * 16, 16)] = res[q]
            obuf_m[pl.ds(q * 16, 16)] = res[8 + q]
        pltpu.sync_copy(obuf_s, osum_hbm.at[g])
        pltpu.sync_copy(obuf_m, omax_hbm.at[g])


def _pre_body(xb, w0, d0b, d1b, bcolb, dis_out, hs_out, cf_out, si_out, ci_out, cacc):
    i = pl.program_id(0)

    @pl.when(i == 0)
    def _():
        cacc[...] = jnp.zeros_like(cacc)

    cnt = d0b[...] + d1b[...]
    dis = lax.rsqrt(cnt + 1.0)
    dis_out[...] = dis
    hs_out[...] = jnp.dot(xb[...], w0[...], preferred_element_type=jnp.float32) * dis
    lanes = lax.broadcasted_iota(jnp.int32, (BLK, G), 1)
    onehot = (bcolb[...] == lanes).astype(jnp.float32)
    cacc[...] += jnp.sum(onehot, axis=0, keepdims=True)

    @pl.when(i == NB - 1)
    def _():
        crow = cacc[...]
        cf_out[...] = crow
        jj = lax.broadcasted_iota(jnp.int32, (G, G), 0)
        gg = lax.broadcasted_iota(jnp.int32, (G, G), 1)
        tri = (jj < gg).astype(jnp.float32)
        starts = jnp.dot(crow, tri, preferred_element_type=jnp.float32)
        si_out[...] = starts.astype(jnp.int32)
        ci_out[...] = crow.astype(jnp.int32)


_pre_call = pl.pallas_call(
    _pre_body,
    grid=(NB,),
    in_specs=[
        pl.BlockSpec((BLK, F), lambda i: (i, 0)),
        pl.BlockSpec((F, F), lambda i: (0, 0)),
        pl.BlockSpec((BLK, 1), lambda i: (i, 0)),
        pl.BlockSpec((BLK, 1), lambda i: (i, 0)),
        pl.BlockSpec((BLK, 1), lambda i: (i, 0)),
    ],
    out_specs=[
        pl.BlockSpec((BLK, 1), lambda i: (i, 0)),
        pl.BlockSpec((BLK, F), lambda i: (i, 0)),
        pl.BlockSpec((1, G), lambda i: (0, 0)),
        pl.BlockSpec((1, G), lambda i: (0, 0)),
        pl.BlockSpec((1, G), lambda i: (0, 0)),
    ],
    out_shape=[
        jax.ShapeDtypeStruct((NPAD, 1), jnp.float32),
        jax.ShapeDtypeStruct((NPAD, F), jnp.float32),
        jax.ShapeDtypeStruct((1, G), jnp.float32),
        jax.ShapeDtypeStruct((1, G), jnp.int32),
        jax.ShapeDtypeStruct((1, G), jnp.int32),
    ],
    scratch_shapes=[pltpu.VMEM((1, G), jnp.float32)],
)


def _make_layer_call(has_next):
    def body(p0b, p1b, hsb, disb, bvec, gvec, bevec, wn, out, tscr, ssum, ssq):
        ph = pl.program_id(0)
        i = pl.program_id(1)
        rows = lax.broadcasted_iota(jnp.int32, (BLK, 1), 0) + i * BLK
        msk = (rows < N).astype(jnp.float32)

        @pl.when(ph == 0)
        def _():
            @pl.when(i == 0)
            def _():
                ssum[...] = jnp.zeros_like(ssum)

            t = disb[...] * (p0b[0] + p1b[0] + hsb[...]) + bvec[...]
            tscr[pl.ds(i * BLK, BLK), :] = t
            ssum[...] += jnp.sum(t * msk, axis=0, keepdims=True)

        @pl.when(ph == 1)
        def _():
            @pl.when(i == 0)
            def _():
                ssq[...] = jnp.zeros_like(ssq)

            m = ssum[...] * (1.0 / N)
            d = (tscr[pl.ds(i * BLK, BLK), :] - m) * msk
            ssq[...] += jnp.sum(d * d, axis=0, keepdims=True)

        @pl.when(ph == 2)
        def _():
            m = ssum[...] * (1.0 / N)
            v = ssq[...] * (1.0 / N)
            t = tscr[pl.ds(i * BLK, BLK), :]
            hb = jnp.maximum((t - m) * lax.rsqrt(v + EPS) * gvec[...] + bevec[...], 0.0) * msk
            if has_next:
                out[...] = jnp.dot(hb, wn[...], preferred_element_type=jnp.float32) * disb[...]
            else:
                out[...] = hb

    return pl.pallas_call(
        body,
        grid=(3, NB),
        in_specs=[
            pl.BlockSpec((1, BLK, F), lambda p, i: (0, i * (p == 0), 0)),
            pl.BlockSpec((1, BLK, F), lambda p, i: (1, i * (p == 0), 0)),
            pl.BlockSpec((BLK, F), lambda p, i: (i * (p == 0), 0)),
            pl.BlockSpec((BLK, 1), lambda p, i: (i, 0)),
            pl.BlockSpec((1, F), lambda p, i: (0, 0)),
            pl.BlockSpec((1, F), lambda p, i: (0, 0)),
            pl.BlockSpec((1, F), lambda p, i: (0, 0)),
            pl.BlockSpec((F, F), lambda p, i: (0, 0)),
        ],
        out_specs=pl.BlockSpec((BLK, F), lambda p, i: (i, 0)),
        out_shape=jax.ShapeDtypeStruct((NPAD, F), jnp.float32),
        scratch_shapes=[pltpu.VMEM((NPAD, F), jnp.float32),
                        pltpu.VMEM((1, F), jnp.float32),
                        pltpu.VMEM((1, F), jnp.float32)],
    )


_layer_call = _make_layer_call(True)
_layer_last_call = _make_layer_call(False)


def _head_body(osum, omax, cfrow, fw1, fb1, fw2, fb2, fw3, fb3, out):
    cnt = cfrow[...]
    rcp = 1.0 / jnp.maximum(cnt, 1.0)
    pos = (cnt > 0.0).astype(jnp.float32)
    ii = lax.broadcasted_iota(jnp.int32, (G, G), 0)
    jj = lax.broadcasted_iota(jnp.int32, (G, G), 1)
    eye = (ii == jj).astype(jnp.float32)
    s = osum[...]
    mean = jnp.dot(eye * rcp, s, preferred_element_type=jnp.float32)
    mx = jnp.dot(eye * pos, omax[...], preferred_element_type=jnp.float32)
    w1 = fw1[...]
    z1 = (jnp.dot(mean, w1[0:G], preferred_element_type=jnp.float32)
          + jnp.dot(mx, w1[G:2 * G], preferred_element_type=jnp.float32)
          + jnp.dot(s, w1[2 * G:3 * G], preferred_element_type=jnp.float32)
          + fb1[...])
    z1 = jnp.maximum(z1, 0.0)
    z2 = jnp.maximum(jnp.dot(z1, fw2[...], preferred_element_type=jnp.float32) + fb2[...], 0.0)
    out[...] = jnp.dot(z2, fw3[...], preferred_element_type=jnp.float32) + fb3[...]


_head_call = pl.pallas_call(
    _head_body,
    out_shape=jax.ShapeDtypeStruct((G, 1), jnp.float32),
)


def kernel(x, edge_index, batch, W0, b0, g0, be0, W1, b1, g1, be1, W2, b2, g2, be2,
           W3, b3, g3, be3, fW1, fb1, fW2, fb2, fW3, fb3):
    xp = jnp.zeros((NPAD, F), jnp.float32).at[:N].set(x)
    pad = N + (jnp.arange(EPAD - E, dtype=jnp.int32) % 128)
    src2d = jnp.concatenate([edge_index[0], pad]).reshape(EROWS, CHK)
    dst2d = jnp.concatenate([edge_index[1], pad]).reshape(EROWS, CHK)
    bcol = jnp.concatenate([batch, jnp.full((NPAD - N,), G, jnp.int32)]).reshape(NPAD, 1)

    degp = _deg_kernel(dst2d)
    d0 = degp[0].reshape(NPAD, 1)
    d1 = degp[1].reshape(NPAD, 1)

    dis, hs, cf, si, ci = _pre_call(xp, W0, d0, d1, bcol)

    layers = [(b0, g0, be0, W1), (b1, g1, be1, W2), (b2, g2, be2, W3), (b3, g3, be3, W3)]
    for i, (bi, gi, bei, wn) in enumerate(layers):
        part = _agg_kernel(hs, src2d, dst2d)
        call = _layer_call if i < 3 else _layer_last_call
        hs = call(part, part, hs, dis,
                  bi.reshape(1, F), gi.reshape(1, F), bei.reshape(1, F), wn)

    si512 = jnp.pad(si.reshape(NW, SEG_PER_W), ((0, 0), (0, 16 - SEG_PER_W))).reshape(NW * 16)
    ci512 = jnp.pad(ci.reshape(NW, SEG_PER_W), ((0, 0), (0, 16 - SEG_PER_W))).reshape(NW * 16)
    osum, omax = _pool_kernel(hs, si512, ci512)
    return _head_call(osum, omax, cf, fW1, fb1.reshape(1, F),
                      fW2, fb2.reshape(1, F // 2), fW3, fb3.reshape(1, 1))
